# baseline (speedup 1.0000x reference)
import functools

import jax
import jax.numpy as jnp
import numpy as np
from jax import lax
from jax.experimental import pallas as pl
from jax.experimental.pallas import tpu as pltpu
from jax.experimental.pallas import tpu_sc as plsc

N_BINS = 15
N_ROWS = 1048576
N_CLASSES = 128
ROWS_PER_BLOCK = 16384
N_BLOCKS = N_ROWS // ROWS_PER_BLOCK
LANE = 128

_EDGES64 = np.linspace(0.0, 1.0, N_BINS + 1)
_LO = _EDGES64.astype(np.float32)

N_TILES = 32
N_PHASES = 4
N_HALF = N_ROWS // N_PHASES
CHUNK = N_HALF // N_TILES
SLICES = CHUNK // 16


def _tc_kernel(logits_ref, labels_ref, conf_ref):
    R = ROWS_PER_BLOCK
    x = logits_ref[...]
    lbl = labels_ref[0, 0, :]

    m = jnp.max(x, axis=1, keepdims=True)
    t = x - m
    e = jnp.exp(t)

    ones_row = jnp.ones((1, N_CLASSES), jnp.float32)
    sT = jax.lax.dot_general(
        ones_row, e, (((1,), (1,)), ((), ())),
        preferred_element_type=jnp.float32)
    iota = jax.lax.broadcasted_iota(jnp.int16, x.shape, 1)
    tb = t.astype(jnp.bfloat16)
    tl = jnp.where(iota == lbl[:, None], tb, jnp.bfloat16(0))
    tlT = jax.lax.dot_general(
        jnp.ones((1, N_CLASSES), jnp.bfloat16), tl,
        (((1,), (1,)), ((), ())),
        preferred_element_type=jnp.float32)

    confT = 1.0 / sT
    conf_ref[...] = jnp.where(tlT == 0.0, -confT, confT).reshape(1, 1, R)


@functools.partial(
    pl.kernel,
    mesh=plsc.VectorSubcoreMesh(core_axis_name="c", subcore_axis_name="s"),
    compiler_params=pltpu.CompilerParams(needs_layout_passes=False),
    out_type=jax.ShapeDtypeStruct((N_TILES, 3 * (N_BINS + 1) * 16), jnp.float32),
    scratch_types=[
        pltpu.VMEM((CHUNK,), jnp.float32),
        pltpu.VMEM((3 * (N_BINS + 1) * 16,), jnp.float32),
    ],
)
def _sc_hist(conf_hbm, out_hbm, conf_v, hist_v):
    wid = lax.axis_index("s") * 2 + lax.axis_index("c")
    base = wid * CHUNK
    pltpu.sync_copy(conf_hbm.at[pl.ds(base, CHUNK)], conf_v)

    zeros16 = jnp.zeros((16,), jnp.float32)
    for s in range(3 * (N_BINS + 1)):
        hist_v[pl.ds(s * 16, 16)] = zeros16

    lane = lax.iota(jnp.int32, 16)
    ones16 = jnp.ones((16,), jnp.float32)
    zero_i = jnp.zeros((16,), jnp.int32)
    one_i = jnp.ones((16,), jnp.int32)
    two_i = one_i + one_i

    zero_f = jnp.zeros((16,), jnp.float32)

    def body(j, carry):
        for u in range(2):
            sv = conf_v[pl.ds((2 * j + u) * 16, 16)]
            cv = jnp.abs(sv)
            av = jnp.where(sv < 0.0, ones16, zero_f)
            k = (cv * np.float32(N_BINS)).astype(jnp.int32)
            flat = k * 16 + lane
            plsc.addupdate_scatter(hist_v, [flat], ones16)
            plsc.addupdate_scatter(hist_v, [flat + 256], cv)
            plsc.addupdate_scatter(hist_v, [flat + 512], av)
        return carry

    lax.fori_loop(0, SLICES // 2, body, 0)

    pltpu.sync_copy(hist_v, out_hbm.at[wid])


def _tc_half(logits, labels3, phase):
    nb = N_HALF // ROWS_PER_BLOCK
    return pl.pallas_call(
        _tc_kernel,
        grid=(nb,),
        in_specs=[
            pl.BlockSpec((ROWS_PER_BLOCK, N_CLASSES),
                         lambda i: (i + phase * nb, 0)),
            pl.BlockSpec((1, 1, ROWS_PER_BLOCK),
                         lambda i: (i + phase * nb, 0, 0)),
        ],
        out_specs=pl.BlockSpec((1, 1, ROWS_PER_BLOCK), lambda i: (i, 0, 0)),
        out_shape=jax.ShapeDtypeStruct((nb, 1, ROWS_PER_BLOCK), jnp.float32),
    )(logits, labels3)


@jax.jit
def _ece(logits, labels):
    labels3 = labels.astype(jnp.int16).reshape(N_BLOCKS, 1, ROWS_PER_BLOCK)
    parts = 0.0
    for p in range(N_PHASES):
        confp = _tc_half(logits, labels3, p)
        parts = parts + _sc_hist(confp.reshape(N_HALF))

    hist = jnp.sum(parts.reshape(N_TILES, 3, N_BINS + 1, 16), axis=(0, 3))
    cnt, sc, sa = hist[0], hist[1], hist[2]
    prop = cnt * (1.0 / N_ROWS)
    safe = jnp.maximum(cnt, 1.0)
    contrib = jnp.abs(sc / safe - sa / safe) * prop
    ece = jnp.sum(jnp.where(prop > 0, contrib, 0.0))
    return ece.reshape(1)


def kernel(logits, labels):
    return _ece(logits, labels)


# speedup vs baseline: 1.0281x; 1.0281x over previous
import functools

import jax
import jax.numpy as jnp
import numpy as np
from jax import lax
from jax.experimental import pallas as pl
from jax.experimental.pallas import tpu as pltpu
from jax.experimental.pallas import tpu_sc as plsc

N_BINS = 15
N_ROWS = 1048576
N_CLASSES = 128
ROWS_PER_BLOCK = 32768
N_BLOCKS = N_ROWS // ROWS_PER_BLOCK
LANE = 128

_EDGES64 = np.linspace(0.0, 1.0, N_BINS + 1)
_LO = _EDGES64.astype(np.float32)

N_TILES = 32
N_HALF = N_ROWS // 2
CHUNK = N_HALF // N_TILES
SLICES = CHUNK // 16


def _tc_kernel(logits_ref, labels_ref, conf_ref):
    R = ROWS_PER_BLOCK
    x = logits_ref[...]
    lbl = labels_ref[0, 0, :]

    m = jnp.max(x, axis=1, keepdims=True)
    t = x - m
    e = jnp.exp(t)

    ones_row = jnp.ones((1, N_CLASSES), jnp.float32)
    sT = jax.lax.dot_general(
        ones_row, e, (((1,), (1,)), ((), ())),
        preferred_element_type=jnp.float32)
    iota = jax.lax.broadcasted_iota(jnp.int16, x.shape, 1)
    tb = t.astype(jnp.bfloat16)
    tl = jnp.where(iota == lbl[:, None], tb, jnp.bfloat16(0))
    tlT = jax.lax.dot_general(
        jnp.ones((1, N_CLASSES), jnp.bfloat16), tl,
        (((1,), (1,)), ((), ())),
        preferred_element_type=jnp.float32)

    confT = 1.0 / sT
    conf_ref[...] = jnp.where(tlT == 0.0, -confT, confT).reshape(1, 1, R)


@functools.partial(
    pl.kernel,
    mesh=plsc.VectorSubcoreMesh(core_axis_name="c", subcore_axis_name="s"),
    compiler_params=pltpu.CompilerParams(needs_layout_passes=False),
    out_type=jax.ShapeDtypeStruct((N_TILES, 3 * (N_BINS + 1) * 16), jnp.float32),
    scratch_types=[
        pltpu.VMEM((CHUNK,), jnp.float32),
        pltpu.VMEM((3 * (N_BINS + 1) * 16,), jnp.float32),
    ],
)
def _sc_hist(conf_hbm, out_hbm, conf_v, hist_v):
    wid = lax.axis_index("s") * 2 + lax.axis_index("c")
    base = wid * CHUNK
    pltpu.sync_copy(conf_hbm.at[pl.ds(base, CHUNK)], conf_v)

    zeros16 = jnp.zeros((16,), jnp.float32)
    for s in range(3 * (N_BINS + 1)):
        hist_v[pl.ds(s * 16, 16)] = zeros16

    lane = lax.iota(jnp.int32, 16)
    ones16 = jnp.ones((16,), jnp.float32)
    zero_i = jnp.zeros((16,), jnp.int32)
    one_i = jnp.ones((16,), jnp.int32)
    two_i = one_i + one_i

    zero_f = jnp.zeros((16,), jnp.float32)

    def body(j, carry):
        for u in range(2):
            sv = conf_v[pl.ds((2 * j + u) * 16, 16)]
            cv = jnp.abs(sv)
            av = jnp.where(sv < 0.0, ones16, zero_f)
            k = (cv * np.float32(N_BINS)).astype(jnp.int32)
            flat = k * 16 + lane
            plsc.addupdate_scatter(hist_v, [flat], ones16)
            plsc.addupdate_scatter(hist_v, [flat + 256], cv)
            plsc.addupdate_scatter(hist_v, [flat + 512], av)
        return carry

    lax.fori_loop(0, SLICES // 2, body, 0)

    pltpu.sync_copy(hist_v, out_hbm.at[wid])


def _tc_half(logits, labels3, phase):
    nb = N_HALF // ROWS_PER_BLOCK
    return pl.pallas_call(
        _tc_kernel,
        grid=(nb,),
        in_specs=[
            pl.BlockSpec((ROWS_PER_BLOCK, N_CLASSES),
                         lambda i: (i + phase * nb, 0)),
            pl.BlockSpec((1, 1, ROWS_PER_BLOCK),
                         lambda i: (i + phase * nb, 0, 0)),
        ],
        out_specs=pl.BlockSpec((1, 1, ROWS_PER_BLOCK), lambda i: (i, 0, 0)),
        out_shape=jax.ShapeDtypeStruct((nb, 1, ROWS_PER_BLOCK), jnp.float32),
    )(logits, labels3)


@jax.jit
def _ece(logits, labels):
    labels3 = labels.astype(jnp.int16).reshape(N_BLOCKS, 1, ROWS_PER_BLOCK)
    conf0 = _tc_half(logits, labels3, 0)
    parts0 = _sc_hist(conf0.reshape(N_HALF))
    conf1 = _tc_half(logits, labels3, 1)
    parts1 = _sc_hist(conf1.reshape(N_HALF))

    parts = parts0 + parts1
    hist = jnp.sum(parts.reshape(N_TILES, 3, N_BINS + 1, 16), axis=(0, 3))
    cnt, sc, sa = hist[0], hist[1], hist[2]
    prop = cnt * (1.0 / N_ROWS)
    safe = jnp.maximum(cnt, 1.0)
    contrib = jnp.abs(sc / safe - sa / safe) * prop
    ece = jnp.sum(jnp.where(prop > 0, contrib, 0.0))
    return ece.reshape(1)


def kernel(logits, labels):
    return _ece(logits, labels)


# speedup vs baseline: 1.0283x; 1.0002x over previous
import functools

import jax
import jax.numpy as jnp
import numpy as np
from jax import lax
from jax.experimental import pallas as pl
from jax.experimental.pallas import tpu as pltpu
from jax.experimental.pallas import tpu_sc as plsc

N_BINS = 15
N_ROWS = 1048576
N_CLASSES = 128
ROWS_PER_BLOCK = 32768
N_BLOCKS = N_ROWS // ROWS_PER_BLOCK
LANE = 128

_EDGES64 = np.linspace(0.0, 1.0, N_BINS + 1)
_LO = _EDGES64.astype(np.float32)

N_TILES = 32
N_HALF = N_ROWS // 2
CHUNK = N_HALF // N_TILES
SLICES = CHUNK // 16


def _tc_kernel(logits_ref, labels_ref, conf_ref):
    R = ROWS_PER_BLOCK
    x = logits_ref[...]
    lbl = labels_ref[0, 0, :]

    m = jnp.max(x, axis=1, keepdims=True)
    t = x - m
    e = jnp.exp(t)

    ones_row = jnp.ones((1, N_CLASSES), jnp.float32)
    sT = jax.lax.dot_general(
        ones_row, e, (((1,), (1,)), ((), ())),
        preferred_element_type=jnp.float32)
    iota = jax.lax.broadcasted_iota(jnp.int16, x.shape, 1)
    tb = t.astype(jnp.bfloat16)
    tl = jnp.where(iota == lbl[:, None], tb, jnp.bfloat16(0))
    tlT = jax.lax.dot_general(
        jnp.ones((1, N_CLASSES), jnp.bfloat16), tl,
        (((1,), (1,)), ((), ())),
        preferred_element_type=jnp.float32)

    confT = 1.0 / sT
    conf_ref[...] = jnp.where(tlT == 0.0, -confT, confT).reshape(1, 1, R)


@functools.partial(
    pl.kernel,
    mesh=plsc.VectorSubcoreMesh(core_axis_name="c", subcore_axis_name="s"),
    compiler_params=pltpu.CompilerParams(needs_layout_passes=False),
    out_type=jax.ShapeDtypeStruct((N_TILES, 3 * (N_BINS + 1) * 16), jnp.float32),
    scratch_types=[
        pltpu.VMEM((CHUNK,), jnp.float32),
        pltpu.VMEM((3 * (N_BINS + 1) * 16,), jnp.float32),
    ],
)
def _sc_hist(conf_hbm, out_hbm, conf_v, hist_v):
    wid = lax.axis_index("s") * 2 + lax.axis_index("c")
    base = wid * CHUNK
    pltpu.sync_copy(conf_hbm.at[pl.ds(base, CHUNK)], conf_v)

    zeros16 = jnp.zeros((16,), jnp.float32)
    for s in range(3 * (N_BINS + 1)):
        hist_v[pl.ds(s * 16, 16)] = zeros16

    lane = lax.iota(jnp.int32, 16)
    ones16 = jnp.ones((16,), jnp.float32)
    zero_i = jnp.zeros((16,), jnp.int32)
    one_i = jnp.ones((16,), jnp.int32)
    two_i = one_i + one_i

    zero_f = jnp.zeros((16,), jnp.float32)

    def body(j, carry):
        for u in range(4):
            sv = conf_v[pl.ds((4 * j + u) * 16, 16)]
            cv = jnp.abs(sv)
            av = jnp.where(sv < 0.0, ones16, zero_f)
            k = (cv * np.float32(N_BINS)).astype(jnp.int32)
            flat = k * 16 + lane
            plsc.addupdate_scatter(hist_v, [flat], ones16)
            plsc.addupdate_scatter(hist_v, [flat + 256], cv)
            plsc.addupdate_scatter(hist_v, [flat + 512], av)
        return carry

    lax.fori_loop(0, SLICES // 4, body, 0)

    pltpu.sync_copy(hist_v, out_hbm.at[wid])


def _tc_half(logits, labels3, phase):
    nb = N_HALF // ROWS_PER_BLOCK
    return pl.pallas_call(
        _tc_kernel,
        grid=(nb,),
        in_specs=[
            pl.BlockSpec((ROWS_PER_BLOCK, N_CLASSES),
                         lambda i: (i + phase * nb, 0)),
            pl.BlockSpec((1, 1, ROWS_PER_BLOCK),
                         lambda i: (i + phase * nb, 0, 0)),
        ],
        out_specs=pl.BlockSpec((1, 1, ROWS_PER_BLOCK), lambda i: (i, 0, 0)),
        out_shape=jax.ShapeDtypeStruct((nb, 1, ROWS_PER_BLOCK), jnp.float32),
    )(logits, labels3)


@jax.jit
def _ece(logits, labels):
    labels3 = labels.astype(jnp.int16).reshape(N_BLOCKS, 1, ROWS_PER_BLOCK)
    conf0 = _tc_half(logits, labels3, 0)
    parts0 = _sc_hist(conf0.reshape(N_HALF))
    conf1 = _tc_half(logits, labels3, 1)
    parts1 = _sc_hist(conf1.reshape(N_HALF))

    parts = parts0 + parts1
    hist = jnp.sum(parts.reshape(N_TILES, 3, N_BINS + 1, 16), axis=(0, 3))
    cnt, sc, sa = hist[0], hist[1], hist[2]
    prop = cnt * (1.0 / N_ROWS)
    safe = jnp.maximum(cnt, 1.0)
    contrib = jnp.abs(sc / safe - sa / safe) * prop
    ece = jnp.sum(jnp.where(prop > 0, contrib, 0.0))
    return ece.reshape(1)


def kernel(logits, labels):
    return _ece(logits, labels)


# speedup vs baseline: 1.0286x; 1.0003x over previous
import functools

import jax
import jax.numpy as jnp
import numpy as np
from jax import lax
from jax.experimental import pallas as pl
from jax.experimental.pallas import tpu as pltpu
from jax.experimental.pallas import tpu_sc as plsc

N_BINS = 15
N_ROWS = 1048576
N_CLASSES = 128
ROWS_PER_BLOCK = 32768
N_BLOCKS = N_ROWS // ROWS_PER_BLOCK

N_TILES = 32
N_HALF = N_ROWS // 2
CHUNK = N_HALF // N_TILES
SLICES = CHUNK // 16


def _tc_kernel(logits_ref, labels_ref, conf_ref):
    R = ROWS_PER_BLOCK
    x = logits_ref[...]
    lbl = labels_ref[0, 0, :]

    m = jnp.max(x, axis=1, keepdims=True)
    t = x - m
    e = jnp.exp(t)

    ones_row = jnp.ones((1, N_CLASSES), jnp.float32)
    sT = jax.lax.dot_general(
        ones_row, e, (((1,), (1,)), ((), ())),
        preferred_element_type=jnp.float32)
    iota = jax.lax.broadcasted_iota(jnp.int16, x.shape, 1)
    tb = t.astype(jnp.bfloat16)
    tl = jnp.where(iota == lbl[:, None], tb, jnp.bfloat16(0))
    tlT = jax.lax.dot_general(
        jnp.ones((1, N_CLASSES), jnp.bfloat16), tl,
        (((1,), (1,)), ((), ())),
        preferred_element_type=jnp.float32)

    confT = 1.0 / sT
    conf_ref[...] = jnp.where(tlT == 0.0, -confT, confT).reshape(1, 1, R)


@functools.partial(
    pl.kernel,
    mesh=plsc.VectorSubcoreMesh(core_axis_name="c", subcore_axis_name="s"),
    compiler_params=pltpu.CompilerParams(needs_layout_passes=False),
    out_type=jax.ShapeDtypeStruct((N_TILES, 3 * (N_BINS + 1) * 16), jnp.float32),
    scratch_types=[
        pltpu.VMEM((CHUNK,), jnp.float32),
        pltpu.VMEM((3 * (N_BINS + 1) * 16,), jnp.float32),
    ],
)
def _sc_hist(conf_hbm, out_hbm, conf_v, hist_v):
    wid = lax.axis_index("s") * 2 + lax.axis_index("c")
    base = wid * CHUNK
    pltpu.sync_copy(conf_hbm.at[pl.ds(base, CHUNK)], conf_v)

    zeros16 = jnp.zeros((16,), jnp.float32)
    for s in range(3 * (N_BINS + 1)):
        hist_v[pl.ds(s * 16, 16)] = zeros16

    lane = lax.iota(jnp.int32, 16)
    ones16 = jnp.ones((16,), jnp.float32)
    zero_f = jnp.zeros((16,), jnp.float32)

    def body(j, carry):
        for u in range(4):
            sv = conf_v[pl.ds((4 * j + u) * 16, 16)]
            cv = jnp.abs(sv)
            av = jnp.where(sv < 0.0, ones16, zero_f)
            k = (cv * np.float32(N_BINS)).astype(jnp.int32)
            flat = k * 16 + lane
            plsc.addupdate_scatter(hist_v, [flat], ones16)
            plsc.addupdate_scatter(hist_v, [flat + 256], cv)
            plsc.addupdate_scatter(hist_v, [flat + 512], av)
        return carry

    lax.fori_loop(0, SLICES // 4, body, 0)

    pltpu.sync_copy(hist_v, out_hbm.at[wid])


def _tc_half(logits, labels3, phase):
    nb = N_HALF // ROWS_PER_BLOCK
    return pl.pallas_call(
        _tc_kernel,
        grid=(nb,),
        in_specs=[
            pl.BlockSpec((ROWS_PER_BLOCK, N_CLASSES),
                         lambda i: (i + phase * nb, 0)),
            pl.BlockSpec((1, 1, ROWS_PER_BLOCK),
                         lambda i: (i + phase * nb, 0, 0)),
        ],
        out_specs=pl.BlockSpec((1, 1, ROWS_PER_BLOCK), lambda i: (i, 0, 0)),
        out_shape=jax.ShapeDtypeStruct((nb, 1, ROWS_PER_BLOCK), jnp.float32),
    )(logits, labels3)


@jax.jit
def _ece(logits, labels):
    labels3 = labels.astype(jnp.int16).reshape(N_BLOCKS, 1, ROWS_PER_BLOCK)
    conf0 = _tc_half(logits, labels3, 0)
    parts0 = _sc_hist(conf0.reshape(N_HALF))
    conf1 = _tc_half(logits, labels3, 1)
    parts1 = _sc_hist(conf1.reshape(N_HALF))

    parts = parts0 + parts1
    hist = jnp.sum(parts.reshape(N_TILES, 3, N_BINS + 1, 16), axis=(0, 3))
    cnt, sc, sa = hist[0], hist[1], hist[2]
    prop = cnt * (1.0 / N_ROWS)
    safe = jnp.maximum(cnt, 1.0)
    contrib = jnp.abs(sc / safe - sa / safe) * prop
    ece = jnp.sum(jnp.where(prop > 0, contrib, 0.0))
    return ece.reshape(1)


def kernel(logits, labels):
    return _ece(logits, labels)
